# Initial kernel scaffold; baseline (speedup 1.0000x reference)
import jax
import jax.numpy as jnp
from jax.experimental import pallas as pl


def kernel(x, prototypes):
    raise NotImplementedError("write your pallas kernel here")


# speedup vs baseline: 9.2068x; 9.2068x over previous
import functools

import jax
import jax.numpy as jnp
from jax import lax
from jax.experimental import pallas as pl
from jax.experimental.pallas import tpu as pltpu
from jax.experimental.pallas import tpu_sc as plsc

B = 256
F = 256
C = 8192
P = 4
KC = 512
JB = 32

NC = 2
NS = 16


def _main_body(x_ref, f2_ref, pt_ref, pred_ref, dst_ref, idx1_ref,
               rmax_ref, ridx_ref):
    k = pl.program_id(0)
    xv = x_ref[...]
    f2 = f2_ref[...]
    nds = []
    for p in range(P):
        ptp = pt_ref[:, p, :]
        cf = lax.dot_general(xv, ptp, (((1,), (1,)), ((), ())),
                             preferred_element_type=jnp.float32)
        c2 = jnp.sum(ptp * ptp, axis=1)
        rst = f2 - 2.0 * cf + c2[None, :]
        nds.append(-rst)
    m01 = jnp.maximum(nds[0], nds[1])
    m23 = jnp.maximum(nds[2], nds[3])
    dstb = jnp.maximum(m01, m23)
    pred_ref[...] = jax.nn.sigmoid(dstb)
    dst_ref[...] = dstb
    rowmax = jnp.max(dstb, axis=1, keepdims=True)
    cols = lax.broadcasted_iota(jnp.int32, dstb.shape, 1) + k * KC
    bidx = jnp.min(jnp.where(dstb == rowmax, cols, jnp.int32(2 ** 30)),
                   axis=1, keepdims=True)

    @pl.when(k == 0)
    def _():
        rmax_ref[...] = rowmax
        ridx_ref[...] = bidx

    @pl.when(k > 0)
    def _():
        better = rowmax > rmax_ref[...]
        ridx_ref[...] = jnp.where(better, bidx, ridx_ref[...])
        rmax_ref[...] = jnp.where(better, rowmax, rmax_ref[...])

    idx1_ref[...] = ridx_ref[...]


def _main_call(x, f2, prototypes):
    return pl.pallas_call(
        _main_body,
        grid=(C // KC,),
        in_specs=[
            pl.BlockSpec((B, F), lambda k: (0, 0)),
            pl.BlockSpec((B, 1), lambda k: (0, 0)),
            pl.BlockSpec((KC, P, F), lambda k: (k, 0, 0)),
        ],
        out_specs=[
            pl.BlockSpec((B, KC), lambda k: (0, k)),
            pl.BlockSpec((B, KC), lambda k: (0, k)),
            pl.BlockSpec((B, 1), lambda k: (0, 0)),
        ],
        out_shape=[
            jax.ShapeDtypeStruct((B, C), jnp.float32),
            jax.ShapeDtypeStruct((B, C), jnp.float32),
            jax.ShapeDtypeStruct((B, 1), jnp.int32),
        ],
        scratch_shapes=[
            pltpu.VMEM((B, 1), jnp.float32),
            pltpu.VMEM((B, 1), jnp.int32),
        ],
    )(x, f2, prototypes)


def _psel_body(idx1_hbm, table_hbm, out_hbm, jv_ref, fi_ref, rows_ref, sem):
    wid = lax.axis_index("s") * NC + lax.axis_index("c")
    base = (wid // 2) * 16
    phalf = (wid % 2) * 2
    pltpu.sync_copy(idx1_hbm.at[pl.ds(base, 16)], jv_ref)
    jv = jv_ref[...]
    for p in (0, 1):
        pp = phalf + p
        fi_ref[...] = jv * P + pp
        pltpu.async_copy(table_hbm.at[fi_ref], rows_ref, sem).wait()
        pltpu.sync_copy(rows_ref, out_hbm.at[pl.ds(pp * B + base, 16)])


@functools.cache
def _psel_call():
    return functools.partial(
        pl.kernel,
        mesh=plsc.VectorSubcoreMesh(core_axis_name="c", subcore_axis_name="s"),
        out_type=jax.ShapeDtypeStruct((P * B, F), jnp.float32),
        scratch_types=[
            pltpu.VMEM((16,), jnp.int32),
            pltpu.VMEM((16,), jnp.int32),
            pltpu.VMEM((16, F), jnp.float32),
            pltpu.SemaphoreType.DMA,
        ],
    )(_psel_body)


def _proto_body(x_ref, f2_ref, psel_ref, proto_ref):
    xv = x_ref[...]
    f2 = f2_ref[...]
    nds, pts = [], []
    for p in range(P):
        ptp = psel_ref[p]
        cf = lax.dot_general(xv, ptp, (((1,), (1,)), ((), ())),
                             preferred_element_type=jnp.float32)
        c2 = jnp.sum(ptp * ptp, axis=1)
        rst = f2 - 2.0 * cf + c2[None, :]
        nds.append(-rst)
        pts.append(ptp)
    m01 = jnp.maximum(nds[0], nds[1])
    g01 = jnp.where(nds[1] > nds[0], jnp.int32(1), jnp.int32(0))
    m23 = jnp.maximum(nds[2], nds[3])
    g23 = jnp.where(nds[3] > nds[2], jnp.int32(3), jnp.int32(2))
    g = jnp.where(m23 > m01, g23, g01)
    for j in range(JB):
        gj = g[:, j:j + 1]
        row = jnp.where(
            gj == 0, pts[0][j:j + 1, :],
            jnp.where(gj == 1, pts[1][j:j + 1, :],
                      jnp.where(gj == 2, pts[2][j:j + 1, :],
                                pts[3][j:j + 1, :])))
        proto_ref[:, j, :] = row


def _proto_call(x, f2, psel):
    return pl.pallas_call(
        _proto_body,
        grid=(B // JB,),
        in_specs=[
            pl.BlockSpec((B, F), lambda k: (0, 0)),
            pl.BlockSpec((B, 1), lambda k: (0, 0)),
            pl.BlockSpec((P, JB, F), lambda k: (0, k, 0)),
        ],
        out_specs=pl.BlockSpec((B, JB, F), lambda k: (0, k, 0)),
        out_shape=jax.ShapeDtypeStruct((B, B, F), jnp.float32),
    )(x, f2, psel)


def kernel(x, prototypes):
    f2 = jnp.sum(x ** 2, axis=1, keepdims=True)
    pred, dst, idx1 = _main_call(x, f2, prototypes)
    table = prototypes.reshape(C * P, F)
    psel = _psel_call()(idx1.reshape(B), table)
    proto = _proto_call(x, f2, psel.reshape(P, B, F))
    return (pred, dst, x, proto)


# speedup vs baseline: 10.2334x; 1.1115x over previous
import functools

import jax
import jax.numpy as jnp
from jax import lax
from jax.experimental import pallas as pl
from jax.experimental.pallas import tpu as pltpu
from jax.experimental.pallas import tpu_sc as plsc

B = 256
F = 256
C = 8192
P = 4
KC = 512
IB = 32

NC = 2
NS = 16


def _main_body(x_ref, f2_ref, pt0_ref, pt1_ref, pt2_ref, pt3_ref,
               pred_ref, dst_ref, idx1_ref, rmax_ref, ridx_ref):
    k = pl.program_id(0)
    xv = x_ref[...]
    f2 = f2_ref[...]
    ones_row = jnp.ones((1, F), jnp.float32)
    nds = []
    for pt_ref in (pt0_ref, pt1_ref, pt2_ref, pt3_ref):
        ptp = pt_ref[...]
        cf = lax.dot_general(xv, ptp, (((1,), (1,)), ((), ())),
                             preferred_element_type=jnp.float32)
        c2row = lax.dot_general(ones_row, ptp * ptp, (((1,), (1,)), ((), ())),
                                preferred_element_type=jnp.float32)
        rst = f2 - 2.0 * cf + c2row
        nds.append(-rst)
    m01 = jnp.maximum(nds[0], nds[1])
    m23 = jnp.maximum(nds[2], nds[3])
    dstb = jnp.maximum(m01, m23)
    pred_ref[...] = jax.nn.sigmoid(dstb)
    dst_ref[...] = dstb
    rowmax = jnp.max(dstb, axis=1, keepdims=True)
    cols = lax.broadcasted_iota(jnp.int32, dstb.shape, 1) + k * KC
    bidx = jnp.min(jnp.where(dstb == rowmax, cols, jnp.int32(2 ** 30)),
                   axis=1, keepdims=True)

    @pl.when(k == 0)
    def _():
        rmax_ref[...] = rowmax
        ridx_ref[...] = bidx

    @pl.when(k > 0)
    def _():
        better = rowmax > rmax_ref[...]
        ridx_ref[...] = jnp.where(better, bidx, ridx_ref[...])
        rmax_ref[...] = jnp.where(better, rowmax, rmax_ref[...])

    idx1_ref[...] = ridx_ref[...]


def _main_call(x, f2, prototypes):
    pt2d = prototypes.reshape(C, P * F)
    return pl.pallas_call(
        _main_body,
        grid=(C // KC,),
        in_specs=[
            pl.BlockSpec((B, F), lambda k: (0, 0)),
            pl.BlockSpec((B, 1), lambda k: (0, 0)),
            pl.BlockSpec((KC, F), lambda k: (k, 0)),
            pl.BlockSpec((KC, F), lambda k: (k, 1)),
            pl.BlockSpec((KC, F), lambda k: (k, 2)),
            pl.BlockSpec((KC, F), lambda k: (k, 3)),
        ],
        out_specs=[
            pl.BlockSpec((B, KC), lambda k: (0, k)),
            pl.BlockSpec((B, KC), lambda k: (0, k)),
            pl.BlockSpec((B, 1), lambda k: (0, 0)),
        ],
        out_shape=[
            jax.ShapeDtypeStruct((B, C), jnp.float32),
            jax.ShapeDtypeStruct((B, C), jnp.float32),
            jax.ShapeDtypeStruct((B, 1), jnp.int32),
        ],
        scratch_shapes=[
            pltpu.VMEM((B, 1), jnp.float32),
            pltpu.VMEM((B, 1), jnp.int32),
        ],
    )(x, f2, pt2d, pt2d, pt2d, pt2d)


def _psel_body(idx1_hbm, table_hbm, out_hbm, jv_ref, fi_ref, rows_ref, sem):
    wid = lax.axis_index("s") * NC + lax.axis_index("c")
    base = (wid // 2) * 16
    phalf = (wid % 2) * 2
    pltpu.sync_copy(idx1_hbm.at[pl.ds(base, 16)], jv_ref)
    jv = jv_ref[...]
    for p in (0, 1):
        pp = phalf + p
        fi_ref[...] = jv * P + pp
        pltpu.async_copy(table_hbm.at[fi_ref], rows_ref, sem).wait()
        pltpu.sync_copy(rows_ref, out_hbm.at[pl.ds(pp * B + base, 16)])


@functools.cache
def _psel_call():
    return functools.partial(
        pl.kernel,
        mesh=plsc.VectorSubcoreMesh(core_axis_name="c", subcore_axis_name="s"),
        out_type=jax.ShapeDtypeStruct((P * B, F), jnp.float32),
        scratch_types=[
            pltpu.VMEM((16,), jnp.int32),
            pltpu.VMEM((16,), jnp.int32),
            pltpu.VMEM((16, F), jnp.float32),
            pltpu.SemaphoreType.DMA,
        ],
    )(_psel_body)


def _proto_body(x_ref, f2t_ref, psel_ref, proto_ref):
    xv = x_ref[...]
    f2t = f2t_ref[0]
    nds, pts = [], []
    for p in range(P):
        ptp = psel_ref[p]
        cft = lax.dot_general(ptp, xv, (((1,), (1,)), ((), ())),
                              preferred_element_type=jnp.float32)
        c2t = jnp.sum(ptp * ptp, axis=1, keepdims=True)
        rst = f2t - 2.0 * cft + c2t
        nds.append(-rst)
        pts.append(ptp)
    m01 = jnp.maximum(nds[0], nds[1])
    g01 = jnp.where(nds[1] > nds[0], jnp.int32(1), jnp.int32(0))
    m23 = jnp.maximum(nds[2], nds[3])
    g23 = jnp.where(nds[3] > nds[2], jnp.int32(3), jnp.int32(2))
    g = jnp.where(m23 > m01, g23, g01)
    for i in range(IB):
        gi = g[:, i:i + 1]
        slab = jnp.where(
            gi == 0, pts[0],
            jnp.where(gi == 1, pts[1],
                      jnp.where(gi == 2, pts[2], pts[3])))
        proto_ref[i * B:(i + 1) * B, :] = slab


def _proto_call(x, f2t, psel):
    return pl.pallas_call(
        _proto_body,
        grid=(B // IB,),
        in_specs=[
            pl.BlockSpec((IB, F), lambda k: (k, 0)),
            pl.BlockSpec((1, 1, IB), lambda k: (k, 0, 0)),
            pl.BlockSpec((P, B, F), lambda k: (0, 0, 0)),
        ],
        out_specs=pl.BlockSpec((IB * B, F), lambda k: (k, 0)),
        out_shape=jax.ShapeDtypeStruct((B * B, F), jnp.float32),
    )(x, f2t, psel)


def kernel(x, prototypes):
    f2 = jnp.sum(x ** 2, axis=1, keepdims=True)
    pred, dst, idx1 = _main_call(x, f2, prototypes)
    table = prototypes.reshape(C * P, F)
    psel = _psel_call()(idx1.reshape(B), table)
    proto = _proto_call(x, f2.reshape(B // IB, 1, IB), psel.reshape(P, B, F))
    return (pred, dst, x, proto.reshape(B, B, F))
